# baseline (speedup 1.0000x reference)
import functools

import jax
import jax.numpy as jnp
from jax import lax
from jax.experimental import pallas as pl
from jax.experimental.pallas import tpu as pltpu
from jax.experimental.pallas import tpu_sc as plsc

L = 128
N = L * L
NUM_CORES = 2
NUM_SUBCORES = 16
NUM_WORKERS = NUM_CORES * NUM_SUBCORES
LANES = 16
CHUNKS = L // LANES
UNROLL = 4


def _make_sc_permute(batch):
    assert batch % (4 * NUM_WORKERS) == 0
    rows_per_worker = batch // NUM_WORKERS
    pairs_per_worker = rows_per_worker // 2

    mesh = plsc.VectorSubcoreMesh(
        core_axis_name="c",
        subcore_axis_name="s",
        num_cores=NUM_CORES,
        num_subcores=NUM_SUBCORES,
    )

    @functools.partial(
        pl.kernel,
        out_type=jax.ShapeDtypeStruct((batch, 1, L, L), jnp.float32),
        mesh=mesh,
        scratch_types=[
            pltpu.VMEM((2, N), jnp.float32),
            pltpu.VMEM((2, N), jnp.float32),
            pltpu.VMEM((1, L, L), jnp.float32),
            pltpu.VMEM((1, L, L), jnp.float32),
            pltpu.SemaphoreType.DMA,
            pltpu.SemaphoreType.DMA,
            pltpu.SemaphoreType.DMA,
            pltpu.SemaphoreType.DMA,
        ],
        compiler_params=pltpu.CompilerParams(needs_layout_passes=False),
    )
    def permute(x_hbm, out_hbm, xin0_v, xin1_v, out0_v, out1_v,
                in_sem0, in_sem1, out_sem0, out_sem1):
        wid = lax.axis_index("s") * NUM_CORES + lax.axis_index("c")
        base = wid * rows_per_worker
        xin_bufs = (xin0_v, xin1_v)
        out_bufs = (out0_v, out1_v)
        in_sems = (in_sem0, in_sem1)
        out_sems = (out_sem0, out_sem1)

        pltpu.async_copy(x_hbm.at[pl.ds(base, 2)], xin0_v, in_sem0)
        pltpu.async_copy(x_hbm.at[pl.ds(base + 2, 2)], xin1_v, in_sem1)

        ii = lax.iota(jnp.int32, LANES)
        col129 = tuple(129 * (u * LANES) + 129 * ii for u in range(CHUNKS))
        thresh = tuple(N + u * LANES + ii for u in range(CHUNKS))
        row_sel = (ii * 0, ii * 0 + 1)

        def do_gather(ip, b2):
            @plsc.parallel_loop(0, L, step=1, unroll=UNROLL)
            def _gather(r):
                rbase = r * L
                for u in range(CHUNKS):
                    flat = rbase + col129[u]
                    idx = jnp.where(flat >= thresh[u], flat - N, flat)
                    out_bufs[b2][0, r, pl.ds(u * LANES, LANES)] = (
                        plsc.load_gather(xin_bufs[ip], [row_sel[b2], idx]))

        def group_body(g, carry):
            for ip in range(2):
                pair = g * 2 + ip
                row0 = base + pair * 2
                pltpu.make_async_copy(
                    x_hbm.at[pl.ds(row0, 2)], xin_bufs[ip],
                    in_sems[ip]).wait()
                for b2 in range(2):
                    row = row0 + b2
                    if ip == 0:
                        @pl.when(g > 0)
                        def _wait_out():
                            pltpu.make_async_copy(
                                out_bufs[b2], out_hbm.at[row - 2],
                                out_sems[b2]).wait()
                    else:
                        pltpu.make_async_copy(
                            out_bufs[b2], out_hbm.at[row - 2],
                            out_sems[b2]).wait()
                    do_gather(ip, b2)
                    pltpu.async_copy(
                        out_bufs[b2], out_hbm.at[row], out_sems[b2])

                @pl.when(row0 + 4 < base + rows_per_worker)
                def _refill():
                    pltpu.async_copy(
                        x_hbm.at[pl.ds(row0 + 4, 2)], xin_bufs[ip],
                        in_sems[ip])
            return carry

        lax.fori_loop(0, pairs_per_worker // 2, group_body, 0, unroll=False)

        last = base + rows_per_worker - 2
        pltpu.make_async_copy(out0_v, out_hbm.at[last], out_sem0).wait()
        pltpu.make_async_copy(out1_v, out_hbm.at[last + 1], out_sem1).wait()

    return permute


def kernel(x):
    batch = x.shape[0]
    x = x.reshape(batch, N)
    return _make_sc_permute(batch)(x)


# speedup vs baseline: 1.1354x; 1.1354x over previous
import functools

import jax
import jax.numpy as jnp
import numpy as np
from jax import lax
from jax.experimental import pallas as pl
from jax.experimental.pallas import tpu as pltpu
from jax.experimental.pallas import tpu_sc as plsc

L = 128
N = L * L
NUM_CORES = 2
NUM_SUBCORES = 16
NUM_WORKERS = NUM_CORES * NUM_SUBCORES
LANES = 16
CHUNKS = L // LANES
UNROLL = 4
DEPTH = 3


def _make_sc_permute(batch):
    assert batch % (2 * NUM_WORKERS) == 0
    rows_per_worker = batch // NUM_WORKERS

    mesh = plsc.VectorSubcoreMesh(
        core_axis_name="c",
        subcore_axis_name="s",
        num_cores=NUM_CORES,
        num_subcores=NUM_SUBCORES,
    )

    @functools.partial(
        pl.kernel,
        out_type=jax.ShapeDtypeStruct((batch, 1, L, L), jnp.float32),
        mesh=mesh,
        scratch_types=[
            pltpu.VMEM((N,), jnp.float32),
            pltpu.VMEM((N,), jnp.float32),
            pltpu.VMEM((N,), jnp.float32),
            pltpu.VMEM((1, L, L), jnp.float32),
            pltpu.VMEM((1, L, L), jnp.float32),
            pltpu.VMEM((1, L, L), jnp.float32),
            pltpu.SemaphoreType.DMA,
            pltpu.SemaphoreType.DMA,
            pltpu.SemaphoreType.DMA,
            pltpu.SemaphoreType.DMA,
            pltpu.SemaphoreType.DMA,
            pltpu.SemaphoreType.DMA,
        ],
        compiler_params=pltpu.CompilerParams(needs_layout_passes=False),
    )
    def permute(x_hbm, out_hbm, xin0_v, xin1_v, xin2_v,
                out0_v, out1_v, out2_v, in_sem0, in_sem1, in_sem2,
                out_sem0, out_sem1, out_sem2):
        wid = lax.axis_index("s") * NUM_CORES + lax.axis_index("c")
        base = wid * rows_per_worker
        xin_bufs = (xin0_v, xin1_v, xin2_v)
        out_bufs = (out0_v, out1_v, out2_v)
        in_sems = (in_sem0, in_sem1, in_sem2)
        out_sems = (out_sem0, out_sem1, out_sem2)

        for b in range(DEPTH):
            pltpu.async_copy(x_hbm.at[base + b], xin_bufs[b], in_sems[b])

        ii = lax.iota(jnp.int32, LANES)
        col129 = tuple(129 * (u * LANES) + 129 * ii for u in range(CHUNKS))
        thresh = tuple(N + u * LANES + ii for u in range(CHUNKS))

        def do_gather(b):
            @plsc.parallel_loop(0, L, step=1, unroll=UNROLL)
            def _gather(r):
                rbase = r * L
                for u in range(CHUNKS):
                    flat = rbase + col129[u]
                    idx = jnp.where(flat >= thresh[u], flat - N, flat)
                    out_bufs[b][0, r, pl.ds(u * LANES, LANES)] = (
                        plsc.load_gather(xin_bufs[b], [idx]))

        groups = rows_per_worker // DEPTH
        rem = rows_per_worker % DEPTH

        def group_body(g, carry):
            for b in range(DEPTH):
                row = base + g * DEPTH + b
                pltpu.make_async_copy(
                    x_hbm.at[row], xin_bufs[b], in_sems[b]).wait()
                @pl.when(g > 0)
                def _wait_out():
                    pltpu.make_async_copy(
                        out_bufs[b], out_hbm.at[row - DEPTH],
                        out_sems[b]).wait()
                do_gather(b)
                pltpu.async_copy(out_bufs[b], out_hbm.at[row], out_sems[b])

                @pl.when(row + DEPTH < base + rows_per_worker)
                def _refill():
                    pltpu.async_copy(
                        x_hbm.at[row + DEPTH], xin_bufs[b], in_sems[b])
            return carry

        lax.fori_loop(0, groups, group_body, 0, unroll=False)

        for j in range(rem):
            b = j
            row = base + groups * DEPTH + j
            pltpu.make_async_copy(
                x_hbm.at[row], xin_bufs[b], in_sems[b]).wait()
            pltpu.make_async_copy(
                out_bufs[b], out_hbm.at[row - DEPTH], out_sems[b]).wait()
            do_gather(b)
            pltpu.async_copy(out_bufs[b], out_hbm.at[row], out_sems[b])

        for k in range(rows_per_worker - DEPTH, rows_per_worker):
            b = k % DEPTH
            pltpu.make_async_copy(
                out_bufs[b], out_hbm.at[base + k], out_sems[b]).wait()

    return permute


def kernel(x):
    batch = x.shape[0]
    x = x.reshape(batch, N)
    return _make_sc_permute(batch)(x)
